# baseline (speedup 1.0000x reference)
import jax
import jax.numpy as jnp
from jax import lax
from jax.experimental import pallas as pl
from jax.experimental.pallas import tpu as pltpu
from jax.experimental.pallas import tpu_sc as plsc

_N = 50000
_E = 800000
_IN = 73
_K = 3
_NC = 2
_NS = 16
_NW = _NC * _NS
_LB = 128
_T0 = 96
_T1 = 304
_TOT = _NS * (_T0 + _T1)
_CHOPS = 16
_EPAD = _TOT * _LB
_DEG_OPS = _TOT // _NW
_DEG_CH = 40
_NP = 50048
_RPS = _NP // _NS
_DW = 8
_TN = 2000


def _mesh():
    return plsc.VectorSubcoreMesh(core_axis_name="c", subcore_axis_name="s",
                                  num_cores=_NC, num_subcores=_NS)


def _deg_body(dstw, ones_hbm, zeros_hbm, out, dst_v, ones_v, acc):
    c = lax.axis_index("c")
    s = lax.axis_index("s")
    wid = s * _NC + c
    r0 = s * _RPS
    pltpu.sync_copy(zeros_hbm, acc.at[pl.ds(r0, _RPS), :])
    pltpu.sync_copy(ones_hbm, ones_v)
    plsc.subcore_barrier()
    base = wid * _DEG_OPS

    def outer(o, carry):
        pltpu.sync_copy(dstw.at[pl.ds(base + o * _DEG_CH, _DEG_CH)], dst_v)

        def body(j, carry2):
            pltpu.sync_copy(ones_v, acc.at[dst_v.at[j]], add=True)
            return carry2

        return lax.fori_loop(0, _DEG_CH, body, carry)

    lax.fori_loop(0, _DEG_OPS // _DEG_CH, outer, 0)
    plsc.subcore_barrier()
    pltpu.sync_copy(acc.at[pl.ds(r0, _RPS), :], out.at[c, pl.ds(r0, _RPS), :])


_deg_call = pl.kernel(
    _deg_body,
    out_type=jax.ShapeDtypeStruct((_NC, _NP, _DW), jnp.float32),
    mesh=_mesh(),
    compiler_params=pltpu.CompilerParams(use_tc_tiling_on_sc=False),
    scratch_types=[
        pltpu.VMEM((_DEG_CH, _LB), jnp.int32),
        pltpu.VMEM((_LB, _DW), jnp.float32),
        pltpu.VMEM_SHARED((_NP, _DW), jnp.float32),
    ],
)


_NBUF = 4
_NGRP = _CHOPS // _NBUF


def _hop_body(u, srcw, dstw, zeros_hbm, out, src_v, dst_v, rows_v, acc, *gsem):
    c = lax.axis_index("c")
    s = lax.axis_index("s")
    r0 = s * _RPS
    pltpu.sync_copy(zeros_hbm, acc.at[pl.ds(r0, _RPS), :])
    plsc.subcore_barrier()
    nchunks = jnp.where(c == 0, _T0 // _CHOPS, _T1 // _CHOPS)
    base = jnp.where(c == 0, s * _T0, _NS * _T0 + s * _T1)

    def outer(o, carry):
        row = base + o * _CHOPS
        pltpu.sync_copy(srcw.at[pl.ds(row, _CHOPS)], src_v)
        pltpu.sync_copy(dstw.at[pl.ds(row, _CHOPS)], dst_v)
        for b in range(_NBUF):
            pltpu.async_copy(u.at[src_v.at[b]], rows_v.at[b], gsem[b])

        def group(g, carry2):
            for b in range(_NBUF):
                j = g * _NBUF + b
                pltpu.make_async_copy(u.at[src_v.at[j]], rows_v.at[b], gsem[b]).wait()
                pltpu.sync_copy(rows_v.at[b], acc.at[dst_v.at[j]], add=True)
                nxt = j + _NBUF

                @pl.when(nxt < _CHOPS)
                def _():
                    pltpu.async_copy(u.at[src_v.at[nxt]], rows_v.at[b], gsem[b])

            return carry2

        return lax.fori_loop(0, _NGRP, group, carry)

    lax.fori_loop(0, nchunks, outer, 0)
    plsc.subcore_barrier()
    pltpu.sync_copy(acc.at[pl.ds(r0, _RPS), :], out.at[c, pl.ds(r0, _RPS), :])


_hop_call = pl.kernel(
    _hop_body,
    out_type=jax.ShapeDtypeStruct((_NC, _NP, 32), jnp.float32),
    mesh=_mesh(),
    compiler_params=pltpu.CompilerParams(use_tc_tiling_on_sc=False),
    scratch_types=[
        pltpu.VMEM((_CHOPS, _LB), jnp.int32),
        pltpu.VMEM((_CHOPS, _LB), jnp.int32),
        pltpu.VMEM((_NBUF, _LB, 32), jnp.float32),
        pltpu.VMEM_SHARED((_NP, 32), jnp.float32),
    ] + [pltpu.SemaphoreType.DMA] * _NBUF,
)


def _pb_kernel(x_ref, w_ref, bt_ref, dp_ref,
               r0_ref, q1_ref, q2_ref, u1_ref, dis_ref, dis2_ref):
    P = jnp.dot(x_ref[...], w_ref[...], preferred_element_type=jnp.float32)
    deg = dp_ref[0, :, 0:1] + dp_ref[1, :, 0:1]
    dis = jnp.where(deg > 0, lax.rsqrt(jnp.maximum(deg, 1.0)), 0.0)
    r0_ref[...] = P[:, 0:32] + bt_ref[...]
    q1_ref[...] = dis * P[:, 32:64]
    q2_ref[...] = dis * P[:, 64:96]
    u1_ref[...] = dis * P[:, 96:128]
    dis_ref[...] = dis
    dis2_ref[...] = dis * dis


def _pb_call(x, wcat, bt2, degp):
    f32 = jnp.float32
    g = _N // _TN
    return pl.pallas_call(
        _pb_kernel,
        grid=(g,),
        in_specs=[
            pl.BlockSpec((_TN, _IN), lambda i: (i, 0)),
            pl.BlockSpec((_IN, 4 * 32), lambda i: (0, 0)),
            pl.BlockSpec((1, 32), lambda i: (0, 0)),
            pl.BlockSpec((_NC, _TN, _DW), lambda i: (0, i, 0)),
        ],
        out_specs=[
            pl.BlockSpec((_TN, 32), lambda i: (i, 0)),
            pl.BlockSpec((_TN, 32), lambda i: (i, 0)),
            pl.BlockSpec((_TN, 32), lambda i: (i, 0)),
            pl.BlockSpec((_TN, 32), lambda i: (i, 0)),
            pl.BlockSpec((_TN, 1), lambda i: (i, 0)),
            pl.BlockSpec((_TN, 1), lambda i: (i, 0)),
        ],
        out_shape=[
            jax.ShapeDtypeStruct((_N, 32), f32),
            jax.ShapeDtypeStruct((_N, 32), f32),
            jax.ShapeDtypeStruct((_N, 32), f32),
            jax.ShapeDtypeStruct((_N, 32), f32),
            jax.ShapeDtypeStruct((_N, 1), f32),
            jax.ShapeDtypeStruct((_N, 1), f32),
        ],
    )(x, wcat, bt2, degp)


def _comb_kernel(pa_ref, dis2_ref, q_ref, out_ref):
    out_ref[...] = dis2_ref[...] * (pa_ref[0] + pa_ref[1]) + q_ref[...]


def _comb_call(partials, dis2, q):
    return pl.pallas_call(
        _comb_kernel,
        grid=(_N // _TN,),
        in_specs=[
            pl.BlockSpec((_NC, _TN, 32), lambda i: (0, i, 0)),
            pl.BlockSpec((_TN, 1), lambda i: (i, 0)),
            pl.BlockSpec((_TN, 32), lambda i: (i, 0)),
        ],
        out_specs=pl.BlockSpec((_TN, 32), lambda i: (i, 0)),
        out_shape=jax.ShapeDtypeStruct((_N, 32), jnp.float32),
    )(partials, dis2, q)


def _acc_stats(i, h, s_ref, q_ref):
    s = jnp.sum(h, axis=0, keepdims=True)
    q = jnp.sum(h * h, axis=0, keepdims=True)

    @pl.when(i == 0)
    def _():
        s_ref[...] = s
        q_ref[...] = q

    @pl.when(i > 0)
    def _():
        s_ref[...] += s
        q_ref[...] += q


def _bn_leaky(h, s, q, g, b):
    m = s * (1.0 / _N)
    v = q * (1.0 / _N) - m * m
    hn = (h - m) * lax.rsqrt(v + 1e-5) * g + b
    return jnp.where(hn >= 0, hn, 0.1 * hn)


def _comb3_kernel(pa_ref, dis_ref, r0_ref, h_ref, s_ref, q_ref):
    i = pl.program_id(0)
    h = dis_ref[...] * (pa_ref[0] + pa_ref[1]) + r0_ref[...]
    h_ref[...] = h
    _acc_stats(i, h, s_ref, q_ref)


def _comb3_call(partials, dis, r0):
    f32 = jnp.float32
    return pl.pallas_call(
        _comb3_kernel,
        grid=(_N // _TN,),
        in_specs=[
            pl.BlockSpec((_NC, _TN, 32), lambda i: (0, i, 0)),
            pl.BlockSpec((_TN, 1), lambda i: (i, 0)),
            pl.BlockSpec((_TN, 32), lambda i: (i, 0)),
        ],
        out_specs=[
            pl.BlockSpec((_TN, 32), lambda i: (i, 0)),
            pl.BlockSpec((1, 32), lambda i: (0, 0)),
            pl.BlockSpec((1, 32), lambda i: (0, 0)),
        ],
        out_shape=[
            jax.ShapeDtypeStruct((_N, 32), f32),
            jax.ShapeDtypeStruct((1, 32), f32),
            jax.ShapeDtypeStruct((1, 32), f32),
        ],
    )(partials, dis, r0)


def _layer_kernel(h_ref, s_ref, q_ref, w_ref, b_ref, g_ref, be_ref,
                  o_ref, so_ref, qo_ref):
    i = pl.program_id(0)
    hn = _bn_leaky(h_ref[...], s_ref[...], q_ref[...], g_ref[...], be_ref[...])
    y = jnp.dot(hn, w_ref[...], preferred_element_type=jnp.float32) + b_ref[...]
    o_ref[...] = y
    _acc_stats(i, y, so_ref, qo_ref)


def _layer_call(h, s, q, w, b, g, be):
    f32 = jnp.float32
    fi, fo = w.shape
    return pl.pallas_call(
        _layer_kernel,
        grid=(_N // _TN,),
        in_specs=[
            pl.BlockSpec((_TN, fi), lambda i: (i, 0)),
            pl.BlockSpec((1, fi), lambda i: (0, 0)),
            pl.BlockSpec((1, fi), lambda i: (0, 0)),
            pl.BlockSpec((fi, fo), lambda i: (0, 0)),
            pl.BlockSpec((1, fo), lambda i: (0, 0)),
            pl.BlockSpec((1, fi), lambda i: (0, 0)),
            pl.BlockSpec((1, fi), lambda i: (0, 0)),
        ],
        out_specs=[
            pl.BlockSpec((_TN, fo), lambda i: (i, 0)),
            pl.BlockSpec((1, fo), lambda i: (0, 0)),
            pl.BlockSpec((1, fo), lambda i: (0, 0)),
        ],
        out_shape=[
            jax.ShapeDtypeStruct((_N, fo), f32),
            jax.ShapeDtypeStruct((1, fo), f32),
            jax.ShapeDtypeStruct((1, fo), f32),
        ],
    )(h, s, q, w, b, g, be)


def _fin_kernel(h_ref, s_ref, q_ref, g_ref, be_ref, out_ref):
    hn = _bn_leaky(h_ref[...], s_ref[...], q_ref[...], g_ref[...], be_ref[...])
    m = jnp.max(hn, axis=1, keepdims=True)
    e = jnp.exp(hn - m)
    out_ref[...] = e / jnp.sum(e, axis=1, keepdims=True)


def _fin_call(h, s, q, g, be):
    return pl.pallas_call(
        _fin_kernel,
        grid=(_N // _TN,),
        in_specs=[
            pl.BlockSpec((_TN, 2), lambda i: (i, 0)),
            pl.BlockSpec((1, 2), lambda i: (0, 0)),
            pl.BlockSpec((1, 2), lambda i: (0, 0)),
            pl.BlockSpec((1, 2), lambda i: (0, 0)),
            pl.BlockSpec((1, 2), lambda i: (0, 0)),
        ],
        out_specs=pl.BlockSpec((_TN, 2), lambda i: (i, 0)),
        out_shape=jax.ShapeDtypeStruct((_N, 2), jnp.float32),
    )(h, s, q, g, be)


def kernel(x, edge_index, W_tag, b_tag, W1, B1, W2, B2, W3, B3, W4, B4,
           g1, be1, g2, be2, g3, be3, g4, be4, g5, be5):
    f32 = jnp.float32
    i32 = jnp.int32
    src = edge_index[0].astype(i32)
    dst = edge_index[1].astype(i32)
    pad = _EPAD - _E
    srcw = jnp.concatenate([src, jnp.zeros((pad,), i32)]).reshape(_TOT, _LB)
    dstw = jnp.concatenate([dst, jnp.full((pad,), _N, i32)]).reshape(_TOT, _LB)
    ones8 = jnp.ones((_LB, _DW), f32)
    zeros8 = jnp.zeros((_RPS, _DW), f32)
    zeros32 = jnp.zeros((_RPS, 32), f32)

    degp = _deg_call(dstw, ones8, zeros8)

    wcat = W_tag.reshape(_K + 1, _IN, 32).transpose(1, 0, 2).reshape(_IN, (_K + 1) * 32)
    bt2 = b_tag.reshape(1, 32)
    r0, q1, q2, u1, dis, dis2 = _pb_call(x, wcat, bt2, degp)

    p1 = _hop_call(u1, srcw, dstw, zeros32)
    u2 = _comb_call(p1, dis2, q2)
    p2 = _hop_call(u2, srcw, dstw, zeros32)
    u3 = _comb_call(p2, dis2, q1)
    p3 = _hop_call(u3, srcw, dstw, zeros32)

    h, s, q = _comb3_call(p3, dis, r0)
    h, s, q = _layer_call(h, s, q, W1, B1.reshape(1, -1), g1.reshape(1, -1), be1.reshape(1, -1))
    h, s, q = _layer_call(h, s, q, W2, B2.reshape(1, -1), g2.reshape(1, -1), be2.reshape(1, -1))
    h, s, q = _layer_call(h, s, q, W3, B3.reshape(1, -1), g3.reshape(1, -1), be3.reshape(1, -1))
    h, s, q = _layer_call(h, s, q, W4, B4.reshape(1, -1), g4.reshape(1, -1), be4.reshape(1, -1))
    return _fin_call(h, s, q, g5.reshape(1, -1), be5.reshape(1, -1))


# speedup vs baseline: 1.1222x; 1.1222x over previous
import jax
import jax.numpy as jnp
from jax import lax
from jax.experimental import pallas as pl
from jax.experimental.pallas import tpu as pltpu
from jax.experimental.pallas import tpu_sc as plsc

_N = 50000
_E = 800000
_IN = 73
_K = 3
_NC = 2
_NS = 16
_NW = _NC * _NS
_LB = 128
_T0 = 288
_T1 = 112
_TOT = _NS * (_T0 + _T1)
_CHOPS = 16
_EPAD = _TOT * _LB
_DEG_OPS = _TOT // _NW
_DEG_CH = 40
_NP = 50048
_RPS = _NP // _NS
_DW = 8
_TN = 2000


def _mesh():
    return plsc.VectorSubcoreMesh(core_axis_name="c", subcore_axis_name="s",
                                  num_cores=_NC, num_subcores=_NS)


def _deg_body(dstw, ones_hbm, zeros_hbm, out, dst_v, ones_v, acc):
    c = lax.axis_index("c")
    s = lax.axis_index("s")
    wid = s * _NC + c
    r0 = s * _RPS
    pltpu.sync_copy(zeros_hbm, acc.at[pl.ds(r0, _RPS), :])
    pltpu.sync_copy(ones_hbm, ones_v)
    plsc.subcore_barrier()
    base = wid * _DEG_OPS

    def outer(o, carry):
        pltpu.sync_copy(dstw.at[pl.ds(base + o * _DEG_CH, _DEG_CH)], dst_v)

        def body(j, carry2):
            pltpu.sync_copy(ones_v, acc.at[dst_v.at[j]], add=True)
            return carry2

        return lax.fori_loop(0, _DEG_CH, body, carry)

    lax.fori_loop(0, _DEG_OPS // _DEG_CH, outer, 0)
    plsc.subcore_barrier()
    pltpu.sync_copy(acc.at[pl.ds(r0, _RPS), :], out.at[c, pl.ds(r0, _RPS), :])


_deg_call = pl.kernel(
    _deg_body,
    out_type=jax.ShapeDtypeStruct((_NC, _NP, _DW), jnp.float32),
    mesh=_mesh(),
    compiler_params=pltpu.CompilerParams(use_tc_tiling_on_sc=False),
    scratch_types=[
        pltpu.VMEM((_DEG_CH, _LB), jnp.int32),
        pltpu.VMEM((_LB, _DW), jnp.float32),
        pltpu.VMEM_SHARED((_NP, _DW), jnp.float32),
    ],
)


_NBUF = 4
_NGRP = _CHOPS // _NBUF


def _hop_body(u, srcw, dstw, zeros_hbm, out, src_v, dst_v, rows_v, acc, *gsem):
    c = lax.axis_index("c")
    s = lax.axis_index("s")
    r0 = s * _RPS
    pltpu.sync_copy(zeros_hbm, acc.at[pl.ds(r0, _RPS), :])
    plsc.subcore_barrier()
    nchunks = jnp.where(c == 0, _T0 // _CHOPS, _T1 // _CHOPS)
    base = jnp.where(c == 0, s * _T0, _NS * _T0 + s * _T1)

    def outer(o, carry):
        row = base + o * _CHOPS
        pltpu.sync_copy(srcw.at[pl.ds(row, _CHOPS)], src_v)
        pltpu.sync_copy(dstw.at[pl.ds(row, _CHOPS)], dst_v)
        for b in range(_NBUF):
            pltpu.async_copy(u.at[src_v.at[b]], rows_v.at[b], gsem[b])

        def group(g, carry2):
            for b in range(_NBUF):
                j = g * _NBUF + b
                pltpu.make_async_copy(u.at[src_v.at[j]], rows_v.at[b], gsem[b]).wait()
                pltpu.sync_copy(rows_v.at[b], acc.at[dst_v.at[j]], add=True)
                nxt = j + _NBUF

                @pl.when(nxt < _CHOPS)
                def _():
                    pltpu.async_copy(u.at[src_v.at[nxt]], rows_v.at[b], gsem[b])

            return carry2

        return lax.fori_loop(0, _NGRP, group, carry)

    lax.fori_loop(0, nchunks, outer, 0)
    plsc.subcore_barrier()
    pltpu.sync_copy(acc.at[pl.ds(r0, _RPS), :], out.at[c, pl.ds(r0, _RPS), :])


_hop_call = pl.kernel(
    _hop_body,
    out_type=jax.ShapeDtypeStruct((_NC, _NP, 32), jnp.float32),
    mesh=_mesh(),
    compiler_params=pltpu.CompilerParams(use_tc_tiling_on_sc=False),
    scratch_types=[
        pltpu.VMEM((_CHOPS, _LB), jnp.int32),
        pltpu.VMEM((_CHOPS, _LB), jnp.int32),
        pltpu.VMEM((_NBUF, _LB, 32), jnp.float32),
        pltpu.VMEM_SHARED((_NP, 32), jnp.float32),
    ] + [pltpu.SemaphoreType.DMA] * _NBUF,
)


def _pb_kernel(x_ref, w_ref, bt_ref, dp_ref,
               r0_ref, q1_ref, q2_ref, u1_ref, dis_ref, dis2_ref):
    P = jnp.dot(x_ref[...], w_ref[...], preferred_element_type=jnp.float32)
    deg = dp_ref[0, :, 0:1] + dp_ref[1, :, 0:1]
    dis = jnp.where(deg > 0, lax.rsqrt(jnp.maximum(deg, 1.0)), 0.0)
    r0_ref[...] = P[:, 0:32] + bt_ref[...]
    q1_ref[...] = dis * P[:, 32:64]
    q2_ref[...] = dis * P[:, 64:96]
    u1_ref[...] = dis * P[:, 96:128]
    dis_ref[...] = dis
    dis2_ref[...] = dis * dis


def _pb_call(x, wcat, bt2, degp):
    f32 = jnp.float32
    g = _N // _TN
    return pl.pallas_call(
        _pb_kernel,
        grid=(g,),
        in_specs=[
            pl.BlockSpec((_TN, _IN), lambda i: (i, 0)),
            pl.BlockSpec((_IN, 4 * 32), lambda i: (0, 0)),
            pl.BlockSpec((1, 32), lambda i: (0, 0)),
            pl.BlockSpec((_NC, _TN, _DW), lambda i: (0, i, 0)),
        ],
        out_specs=[
            pl.BlockSpec((_TN, 32), lambda i: (i, 0)),
            pl.BlockSpec((_TN, 32), lambda i: (i, 0)),
            pl.BlockSpec((_TN, 32), lambda i: (i, 0)),
            pl.BlockSpec((_TN, 32), lambda i: (i, 0)),
            pl.BlockSpec((_TN, 1), lambda i: (i, 0)),
            pl.BlockSpec((_TN, 1), lambda i: (i, 0)),
        ],
        out_shape=[
            jax.ShapeDtypeStruct((_N, 32), f32),
            jax.ShapeDtypeStruct((_N, 32), f32),
            jax.ShapeDtypeStruct((_N, 32), f32),
            jax.ShapeDtypeStruct((_N, 32), f32),
            jax.ShapeDtypeStruct((_N, 1), f32),
            jax.ShapeDtypeStruct((_N, 1), f32),
        ],
    )(x, wcat, bt2, degp)


def _comb_kernel(pa_ref, dis2_ref, q_ref, out_ref):
    out_ref[...] = dis2_ref[...] * (pa_ref[0] + pa_ref[1]) + q_ref[...]


def _comb_call(partials, dis2, q):
    return pl.pallas_call(
        _comb_kernel,
        grid=(_N // _TN,),
        in_specs=[
            pl.BlockSpec((_NC, _TN, 32), lambda i: (0, i, 0)),
            pl.BlockSpec((_TN, 1), lambda i: (i, 0)),
            pl.BlockSpec((_TN, 32), lambda i: (i, 0)),
        ],
        out_specs=pl.BlockSpec((_TN, 32), lambda i: (i, 0)),
        out_shape=jax.ShapeDtypeStruct((_N, 32), jnp.float32),
    )(partials, dis2, q)


def _acc_stats(i, h, s_ref, q_ref):
    s = jnp.sum(h, axis=0, keepdims=True)
    q = jnp.sum(h * h, axis=0, keepdims=True)

    @pl.when(i == 0)
    def _():
        s_ref[...] = s
        q_ref[...] = q

    @pl.when(i > 0)
    def _():
        s_ref[...] += s
        q_ref[...] += q


def _bn_leaky(h, s, q, g, b):
    m = s * (1.0 / _N)
    v = q * (1.0 / _N) - m * m
    hn = (h - m) * lax.rsqrt(v + 1e-5) * g + b
    return jnp.where(hn >= 0, hn, 0.1 * hn)


def _comb3_kernel(pa_ref, dis_ref, r0_ref, h_ref, s_ref, q_ref):
    i = pl.program_id(0)
    h = dis_ref[...] * (pa_ref[0] + pa_ref[1]) + r0_ref[...]
    h_ref[...] = h
    _acc_stats(i, h, s_ref, q_ref)


def _comb3_call(partials, dis, r0):
    f32 = jnp.float32
    return pl.pallas_call(
        _comb3_kernel,
        grid=(_N // _TN,),
        in_specs=[
            pl.BlockSpec((_NC, _TN, 32), lambda i: (0, i, 0)),
            pl.BlockSpec((_TN, 1), lambda i: (i, 0)),
            pl.BlockSpec((_TN, 32), lambda i: (i, 0)),
        ],
        out_specs=[
            pl.BlockSpec((_TN, 32), lambda i: (i, 0)),
            pl.BlockSpec((1, 32), lambda i: (0, 0)),
            pl.BlockSpec((1, 32), lambda i: (0, 0)),
        ],
        out_shape=[
            jax.ShapeDtypeStruct((_N, 32), f32),
            jax.ShapeDtypeStruct((1, 32), f32),
            jax.ShapeDtypeStruct((1, 32), f32),
        ],
    )(partials, dis, r0)


def _layer_kernel(h_ref, s_ref, q_ref, w_ref, b_ref, g_ref, be_ref,
                  o_ref, so_ref, qo_ref):
    i = pl.program_id(0)
    hn = _bn_leaky(h_ref[...], s_ref[...], q_ref[...], g_ref[...], be_ref[...])
    y = jnp.dot(hn, w_ref[...], preferred_element_type=jnp.float32) + b_ref[...]
    o_ref[...] = y
    _acc_stats(i, y, so_ref, qo_ref)


def _layer_call(h, s, q, w, b, g, be):
    f32 = jnp.float32
    fi, fo = w.shape
    return pl.pallas_call(
        _layer_kernel,
        grid=(_N // _TN,),
        in_specs=[
            pl.BlockSpec((_TN, fi), lambda i: (i, 0)),
            pl.BlockSpec((1, fi), lambda i: (0, 0)),
            pl.BlockSpec((1, fi), lambda i: (0, 0)),
            pl.BlockSpec((fi, fo), lambda i: (0, 0)),
            pl.BlockSpec((1, fo), lambda i: (0, 0)),
            pl.BlockSpec((1, fi), lambda i: (0, 0)),
            pl.BlockSpec((1, fi), lambda i: (0, 0)),
        ],
        out_specs=[
            pl.BlockSpec((_TN, fo), lambda i: (i, 0)),
            pl.BlockSpec((1, fo), lambda i: (0, 0)),
            pl.BlockSpec((1, fo), lambda i: (0, 0)),
        ],
        out_shape=[
            jax.ShapeDtypeStruct((_N, fo), f32),
            jax.ShapeDtypeStruct((1, fo), f32),
            jax.ShapeDtypeStruct((1, fo), f32),
        ],
    )(h, s, q, w, b, g, be)


def _fin_kernel(h_ref, s_ref, q_ref, g_ref, be_ref, out_ref):
    hn = _bn_leaky(h_ref[...], s_ref[...], q_ref[...], g_ref[...], be_ref[...])
    m = jnp.max(hn, axis=1, keepdims=True)
    e = jnp.exp(hn - m)
    out_ref[...] = e / jnp.sum(e, axis=1, keepdims=True)


def _fin_call(h, s, q, g, be):
    return pl.pallas_call(
        _fin_kernel,
        grid=(_N // _TN,),
        in_specs=[
            pl.BlockSpec((_TN, 2), lambda i: (i, 0)),
            pl.BlockSpec((1, 2), lambda i: (0, 0)),
            pl.BlockSpec((1, 2), lambda i: (0, 0)),
            pl.BlockSpec((1, 2), lambda i: (0, 0)),
            pl.BlockSpec((1, 2), lambda i: (0, 0)),
        ],
        out_specs=pl.BlockSpec((_TN, 2), lambda i: (i, 0)),
        out_shape=jax.ShapeDtypeStruct((_N, 2), jnp.float32),
    )(h, s, q, g, be)


def kernel(x, edge_index, W_tag, b_tag, W1, B1, W2, B2, W3, B3, W4, B4,
           g1, be1, g2, be2, g3, be3, g4, be4, g5, be5):
    f32 = jnp.float32
    i32 = jnp.int32
    src = edge_index[0].astype(i32)
    dst = edge_index[1].astype(i32)
    pad = _EPAD - _E
    srcw = jnp.concatenate([src, jnp.zeros((pad,), i32)]).reshape(_TOT, _LB)
    dstw = jnp.concatenate([dst, jnp.full((pad,), _N, i32)]).reshape(_TOT, _LB)
    ones8 = jnp.ones((_LB, _DW), f32)
    zeros8 = jnp.zeros((_RPS, _DW), f32)
    zeros32 = jnp.zeros((_RPS, 32), f32)

    degp = _deg_call(dstw, ones8, zeros8)

    wcat = W_tag.reshape(_K + 1, _IN, 32).transpose(1, 0, 2).reshape(_IN, (_K + 1) * 32)
    bt2 = b_tag.reshape(1, 32)
    r0, q1, q2, u1, dis, dis2 = _pb_call(x, wcat, bt2, degp)

    p1 = _hop_call(u1, srcw, dstw, zeros32)
    u2 = _comb_call(p1, dis2, q2)
    p2 = _hop_call(u2, srcw, dstw, zeros32)
    u3 = _comb_call(p2, dis2, q1)
    p3 = _hop_call(u3, srcw, dstw, zeros32)

    h, s, q = _comb3_call(p3, dis, r0)
    h, s, q = _layer_call(h, s, q, W1, B1.reshape(1, -1), g1.reshape(1, -1), be1.reshape(1, -1))
    h, s, q = _layer_call(h, s, q, W2, B2.reshape(1, -1), g2.reshape(1, -1), be2.reshape(1, -1))
    h, s, q = _layer_call(h, s, q, W3, B3.reshape(1, -1), g3.reshape(1, -1), be3.reshape(1, -1))
    h, s, q = _layer_call(h, s, q, W4, B4.reshape(1, -1), g4.reshape(1, -1), be4.reshape(1, -1))
    return _fin_call(h, s, q, g5.reshape(1, -1), be5.reshape(1, -1))


# speedup vs baseline: 1.5863x; 1.4135x over previous
import jax
import jax.numpy as jnp
from jax import lax
from jax.experimental import pallas as pl
from jax.experimental.pallas import tpu as pltpu
from jax.experimental.pallas import tpu_sc as plsc

_N = 50000
_E = 800000
_IN = 73
_K = 3
_NC = 2
_NS = 16
_NW = _NC * _NS
_LB = 128
_TOT = 6400
_SOPS = _TOT // _NS
_CHOPS = 16
_EPAD = _TOT * _LB
_DEG_OPS = _TOT // _NW
_DEG_CH = 40
_RCH = 184
_NP = 50048
_RPS = _NP // _NS
_DW = 8
_TN = 2000


def _mesh():
    return plsc.VectorSubcoreMesh(core_axis_name="c", subcore_axis_name="s",
                                  num_cores=_NC, num_subcores=_NS)


def _deg_body(dstw, ones_hbm, zeros_hbm, out, dst_v, ones_v, acc):
    c = lax.axis_index("c")
    s = lax.axis_index("s")
    wid = s * _NC + c
    r0 = s * _RPS
    pltpu.sync_copy(zeros_hbm, acc.at[pl.ds(r0, _RPS), :])
    pltpu.sync_copy(ones_hbm, ones_v)
    plsc.subcore_barrier()
    base = wid * _DEG_OPS

    def outer(o, carry):
        pltpu.sync_copy(dstw.at[pl.ds(base + o * _DEG_CH, _DEG_CH)], dst_v)

        def body(j, carry2):
            pltpu.sync_copy(ones_v, acc.at[dst_v.at[j]], add=True)
            return carry2

        return lax.fori_loop(0, _DEG_CH, body, carry)

    lax.fori_loop(0, _DEG_OPS // _DEG_CH, outer, 0)
    plsc.subcore_barrier()
    pltpu.sync_copy(acc.at[pl.ds(r0, _RPS), :], out.at[c, pl.ds(r0, _RPS), :])


_deg_call = pl.kernel(
    _deg_body,
    out_type=jax.ShapeDtypeStruct((_NC, _NP, _DW), jnp.float32),
    mesh=_mesh(),
    compiler_params=pltpu.CompilerParams(use_tc_tiling_on_sc=False),
    scratch_types=[
        pltpu.VMEM((_DEG_CH, _LB), jnp.int32),
        pltpu.VMEM((_LB, _DW), jnp.float32),
        pltpu.VMEM_SHARED((_NP, _DW), jnp.float32),
    ],
)


_NBUF = 4
_NGRP = _CHOPS // _NBUF


def _mega_body(u1, srcw, dstw, q, dis2, zeros16, out,
               src_v, dst_v, rows_v, cb_a, cb_q, cb_u, cb_d, acc, uS, *gsem):
    c = lax.axis_index("c")
    s = lax.axis_index("s")
    r0 = s * _RPS
    base = s * _SOPS

    def scatter_phase(table):
        def outer(o, carry):
            row = base + o * _CHOPS
            pltpu.sync_copy(srcw.at[pl.ds(row, _CHOPS)], src_v)
            pltpu.sync_copy(dstw.at[pl.ds(row, _CHOPS)], dst_v)
            for b in range(_NBUF):
                pltpu.async_copy(table.at[src_v.at[b]], rows_v.at[b], gsem[b])

            def group(g, carry2):
                for b in range(_NBUF):
                    j = g * _NBUF + b
                    pltpu.make_async_copy(table.at[src_v.at[j]], rows_v.at[b], gsem[b]).wait()
                    pltpu.sync_copy(rows_v.at[b], acc.at[dst_v.at[j]], add=True)
                    nxt = j + _NBUF

                    @pl.when(nxt < _CHOPS)
                    def _():
                        pltpu.async_copy(table.at[src_v.at[nxt]], rows_v.at[b], gsem[b])

                return carry2

            return lax.fori_loop(0, _NGRP, group, carry)

        lax.fori_loop(0, _SOPS // _CHOPS, outer, 0)

    def combine_phase(qoff):
        def chunk(k, carry):
            row = r0 + k * _RCH
            pltpu.sync_copy(acc.at[pl.ds(row, _RCH), :], cb_a)
            pltpu.sync_copy(q.at[pl.ds(row, _RCH), pl.ds(qoff, 16)], cb_q)
            pltpu.sync_copy(dis2.at[pl.ds(row, _RCH)], cb_d)

            def rowloop(r, carry2):
                d = plsc.load_gather(cb_d, [jnp.full((16,), r, jnp.int32)])
                cb_u[r, :] = d * cb_a[r, :] + cb_q[r, :]
                return carry2

            lax.fori_loop(0, _RCH, rowloop, carry)
            pltpu.sync_copy(cb_u, uS.at[pl.ds(row, _RCH), :])
            return carry

        lax.fori_loop(0, _RPS // _RCH, chunk, 0)
        pltpu.sync_copy(zeros16, acc.at[pl.ds(r0, _RPS), :])

    qc = 16 * c
    pltpu.sync_copy(zeros16, acc.at[pl.ds(r0, _RPS), :])
    plsc.subcore_barrier()
    scatter_phase(u1.at[c])
    plsc.subcore_barrier()
    combine_phase(qc)
    plsc.subcore_barrier()
    scatter_phase(uS)
    plsc.subcore_barrier()
    combine_phase(32 + qc)
    plsc.subcore_barrier()
    scatter_phase(uS)
    plsc.subcore_barrier()
    pltpu.sync_copy(acc.at[pl.ds(r0, _RPS), :], out.at[pl.ds(r0, _RPS), pl.ds(qc, 16)])


_mega_call = pl.kernel(
    _mega_body,
    out_type=jax.ShapeDtypeStruct((_NP, 32), jnp.float32),
    mesh=_mesh(),
    compiler_params=pltpu.CompilerParams(use_tc_tiling_on_sc=False,
                                         needs_layout_passes=False),
    scratch_types=[
        pltpu.VMEM((_CHOPS, _LB), jnp.int32),
        pltpu.VMEM((_CHOPS, _LB), jnp.int32),
        pltpu.VMEM((_NBUF, _LB, 16), jnp.float32),
        pltpu.VMEM((_RCH, 16), jnp.float32),
        pltpu.VMEM((_RCH, 16), jnp.float32),
        pltpu.VMEM((_RCH, 16), jnp.float32),
        pltpu.VMEM((_RCH,), jnp.float32),
        pltpu.VMEM_SHARED((_NP, 16), jnp.float32),
        pltpu.VMEM_SHARED((_NP, 16), jnp.float32),
    ] + [pltpu.SemaphoreType.DMA] * _NBUF,
)


def _pb_kernel(x_ref, w_ref, bt_ref, dp_ref,
               r0_ref, q_ref, u1_ref, dis_ref, dis2_ref):
    P = jnp.dot(x_ref[...], w_ref[...], preferred_element_type=jnp.float32)
    deg = dp_ref[0, :, 0:1] + dp_ref[1, :, 0:1]
    dis = jnp.where(deg > 0, lax.rsqrt(jnp.maximum(deg, 1.0)), 0.0)
    r0_ref[...] = P[:, 0:32] + bt_ref[...]
    q_ref[:, 0:32] = dis * P[:, 64:96]
    q_ref[:, 32:64] = dis * P[:, 32:64]
    u1 = dis * P[:, 96:128]
    u1_ref[0] = u1[:, 0:16]
    u1_ref[1] = u1[:, 16:32]
    dis_ref[...] = dis
    dis2_ref[...] = dis * dis


def _pb_call(x, wcat, bt2, degp):
    f32 = jnp.float32
    g = _N // _TN
    return pl.pallas_call(
        _pb_kernel,
        grid=(g,),
        in_specs=[
            pl.BlockSpec((_TN, _IN), lambda i: (i, 0)),
            pl.BlockSpec((_IN, 4 * 32), lambda i: (0, 0)),
            pl.BlockSpec((1, 32), lambda i: (0, 0)),
            pl.BlockSpec((_NC, _TN, _DW), lambda i: (0, i, 0)),
        ],
        out_specs=[
            pl.BlockSpec((_TN, 32), lambda i: (i, 0)),
            pl.BlockSpec((_TN, 64), lambda i: (i, 0)),
            pl.BlockSpec((_NC, _TN, 16), lambda i: (0, i, 0)),
            pl.BlockSpec((_TN, 1), lambda i: (i, 0)),
            pl.BlockSpec((_TN, 1), lambda i: (i, 0)),
        ],
        out_shape=[
            jax.ShapeDtypeStruct((_N, 32), f32),
            jax.ShapeDtypeStruct((_N, 64), f32),
            jax.ShapeDtypeStruct((_NC, _N, 16), f32),
            jax.ShapeDtypeStruct((_N, 1), f32),
            jax.ShapeDtypeStruct((_N, 1), f32),
        ],
    )(x, wcat, bt2, degp)


def _acc_stats(i, h, s_ref, q_ref):
    s = jnp.sum(h, axis=0, keepdims=True)
    q = jnp.sum(h * h, axis=0, keepdims=True)

    @pl.when(i == 0)
    def _():
        s_ref[...] = s
        q_ref[...] = q

    @pl.when(i > 0)
    def _():
        s_ref[...] += s
        q_ref[...] += q


def _bn_leaky(h, s, q, g, b):
    m = s * (1.0 / _N)
    v = q * (1.0 / _N) - m * m
    hn = (h - m) * lax.rsqrt(v + 1e-5) * g + b
    return jnp.where(hn >= 0, hn, 0.1 * hn)


def _comb3_kernel(pa_ref, dis_ref, r0_ref, h_ref, s_ref, q_ref):
    i = pl.program_id(0)
    h = dis_ref[...] * pa_ref[...] + r0_ref[...]
    h_ref[...] = h
    _acc_stats(i, h, s_ref, q_ref)


def _comb3_call(h3, dis, r0):
    f32 = jnp.float32
    return pl.pallas_call(
        _comb3_kernel,
        grid=(_N // _TN,),
        in_specs=[
            pl.BlockSpec((_TN, 32), lambda i: (i, 0)),
            pl.BlockSpec((_TN, 1), lambda i: (i, 0)),
            pl.BlockSpec((_TN, 32), lambda i: (i, 0)),
        ],
        out_specs=[
            pl.BlockSpec((_TN, 32), lambda i: (i, 0)),
            pl.BlockSpec((1, 32), lambda i: (0, 0)),
            pl.BlockSpec((1, 32), lambda i: (0, 0)),
        ],
        out_shape=[
            jax.ShapeDtypeStruct((_N, 32), f32),
            jax.ShapeDtypeStruct((1, 32), f32),
            jax.ShapeDtypeStruct((1, 32), f32),
        ],
    )(h3, dis, r0)


def _layer_kernel(h_ref, s_ref, q_ref, w_ref, b_ref, g_ref, be_ref,
                  o_ref, so_ref, qo_ref):
    i = pl.program_id(0)
    hn = _bn_leaky(h_ref[...], s_ref[...], q_ref[...], g_ref[...], be_ref[...])
    y = jnp.dot(hn, w_ref[...], preferred_element_type=jnp.float32) + b_ref[...]
    o_ref[...] = y
    _acc_stats(i, y, so_ref, qo_ref)


def _layer_call(h, s, q, w, b, g, be):
    f32 = jnp.float32
    fi, fo = w.shape
    return pl.pallas_call(
        _layer_kernel,
        grid=(_N // _TN,),
        in_specs=[
            pl.BlockSpec((_TN, fi), lambda i: (i, 0)),
            pl.BlockSpec((1, fi), lambda i: (0, 0)),
            pl.BlockSpec((1, fi), lambda i: (0, 0)),
            pl.BlockSpec((fi, fo), lambda i: (0, 0)),
            pl.BlockSpec((1, fo), lambda i: (0, 0)),
            pl.BlockSpec((1, fi), lambda i: (0, 0)),
            pl.BlockSpec((1, fi), lambda i: (0, 0)),
        ],
        out_specs=[
            pl.BlockSpec((_TN, fo), lambda i: (i, 0)),
            pl.BlockSpec((1, fo), lambda i: (0, 0)),
            pl.BlockSpec((1, fo), lambda i: (0, 0)),
        ],
        out_shape=[
            jax.ShapeDtypeStruct((_N, fo), f32),
            jax.ShapeDtypeStruct((1, fo), f32),
            jax.ShapeDtypeStruct((1, fo), f32),
        ],
    )(h, s, q, w, b, g, be)


def _fin_kernel(h_ref, s_ref, q_ref, g_ref, be_ref, out_ref):
    hn = _bn_leaky(h_ref[...], s_ref[...], q_ref[...], g_ref[...], be_ref[...])
    m = jnp.max(hn, axis=1, keepdims=True)
    e = jnp.exp(hn - m)
    out_ref[...] = e / jnp.sum(e, axis=1, keepdims=True)


def _fin_call(h, s, q, g, be):
    return pl.pallas_call(
        _fin_kernel,
        grid=(_N // _TN,),
        in_specs=[
            pl.BlockSpec((_TN, 2), lambda i: (i, 0)),
            pl.BlockSpec((1, 2), lambda i: (0, 0)),
            pl.BlockSpec((1, 2), lambda i: (0, 0)),
            pl.BlockSpec((1, 2), lambda i: (0, 0)),
            pl.BlockSpec((1, 2), lambda i: (0, 0)),
        ],
        out_specs=pl.BlockSpec((_TN, 2), lambda i: (i, 0)),
        out_shape=jax.ShapeDtypeStruct((_N, 2), jnp.float32),
    )(h, s, q, g, be)


def kernel(x, edge_index, W_tag, b_tag, W1, B1, W2, B2, W3, B3, W4, B4,
           g1, be1, g2, be2, g3, be3, g4, be4, g5, be5):
    f32 = jnp.float32
    i32 = jnp.int32
    src = edge_index[0].astype(i32)
    dst = edge_index[1].astype(i32)
    pad = _EPAD - _E
    srcw = jnp.concatenate([src, jnp.zeros((pad,), i32)]).reshape(_TOT, _LB)
    dstw = jnp.concatenate([dst, jnp.full((pad,), _N, i32)]).reshape(_TOT, _LB)
    ones8 = jnp.ones((_LB, _DW), f32)
    zeros8 = jnp.zeros((_RPS, _DW), f32)
    zeros16 = jnp.zeros((_RPS, 16), f32)

    degp = _deg_call(dstw, ones8, zeros8)

    wcat = W_tag.reshape(_K + 1, _IN, 32).transpose(1, 0, 2).reshape(_IN, (_K + 1) * 32)
    bt2 = b_tag.reshape(1, 32)
    r0, qq, u1, dis, dis2 = _pb_call(x, wcat, bt2, degp)

    h3 = _mega_call(u1, srcw, dstw, qq, dis2.reshape(_N), zeros16)

    h, s, q = _comb3_call(h3, dis, r0)
    h, s, q = _layer_call(h, s, q, W1, B1.reshape(1, -1), g1.reshape(1, -1), be1.reshape(1, -1))
    h, s, q = _layer_call(h, s, q, W2, B2.reshape(1, -1), g2.reshape(1, -1), be2.reshape(1, -1))
    h, s, q = _layer_call(h, s, q, W3, B3.reshape(1, -1), g3.reshape(1, -1), be3.reshape(1, -1))
    h, s, q = _layer_call(h, s, q, W4, B4.reshape(1, -1), g4.reshape(1, -1), be4.reshape(1, -1))
    return _fin_call(h, s, q, g5.reshape(1, -1), be5.reshape(1, -1))


# speedup vs baseline: 1.8028x; 1.1365x over previous
import jax
import jax.numpy as jnp
from jax import lax
from jax.experimental import pallas as pl
from jax.experimental.pallas import tpu as pltpu
from jax.experimental.pallas import tpu_sc as plsc

_N = 50000
_E = 800000
_IN = 73
_K = 3
_NC = 2
_NS = 16
_NW = _NC * _NS
_LB = 128
_TOT = 6400
_SOPS = _TOT // _NS
_CHOPS = 16
_EPAD = _TOT * _LB
_DEG_OPS = _TOT // _NW
_DEG_CH = 40
_RCH = 184
_NP = 50048
_RPS = _NP // _NS
_DW = 8
_TN = 2000


def _mesh():
    return plsc.VectorSubcoreMesh(core_axis_name="c", subcore_axis_name="s",
                                  num_cores=_NC, num_subcores=_NS)


def _deg_body(dstw, ones_hbm, zeros_hbm, out, dst_v, ones_v, acc):
    c = lax.axis_index("c")
    s = lax.axis_index("s")
    wid = s * _NC + c
    r0 = s * _RPS
    pltpu.sync_copy(zeros_hbm, acc.at[pl.ds(r0, _RPS), :])
    pltpu.sync_copy(ones_hbm, ones_v)
    plsc.subcore_barrier()
    base = wid * _DEG_OPS

    def outer(o, carry):
        pltpu.sync_copy(dstw.at[pl.ds(base + o * _DEG_CH, _DEG_CH)], dst_v)

        def body(j, carry2):
            pltpu.sync_copy(ones_v, acc.at[dst_v.at[j]], add=True)
            return carry2

        return lax.fori_loop(0, _DEG_CH, body, carry)

    lax.fori_loop(0, _DEG_OPS // _DEG_CH, outer, 0)
    plsc.subcore_barrier()
    pltpu.sync_copy(acc.at[pl.ds(r0, _RPS), :], out.at[c, pl.ds(r0, _RPS), :])


_deg_call = pl.kernel(
    _deg_body,
    out_type=jax.ShapeDtypeStruct((_NC, _NP, _DW), jnp.float32),
    mesh=_mesh(),
    compiler_params=pltpu.CompilerParams(use_tc_tiling_on_sc=False),
    scratch_types=[
        pltpu.VMEM((_DEG_CH, _LB), jnp.int32),
        pltpu.VMEM((_LB, _DW), jnp.float32),
        pltpu.VMEM_SHARED((_NP, _DW), jnp.float32),
    ],
)


_NBUF = 4
_NGRP = _CHOPS // _NBUF


def _mega_body(u1, srcw, dstw, q, dis2, zeros16, out,
               src_v, dst_v, rows_v, cb_a, cb_q, cb_u, cb_d, acc, uS, *gsem):
    c = lax.axis_index("c")
    s = lax.axis_index("s")
    r0 = s * _RPS
    base = s * _SOPS

    def scatter_phase(table):
        def outer(o, carry):
            row = base + o * _CHOPS
            pltpu.sync_copy(srcw.at[pl.ds(row, _CHOPS)], src_v)
            pltpu.sync_copy(dstw.at[pl.ds(row, _CHOPS)], dst_v)
            for b in range(_NBUF):
                pltpu.async_copy(table.at[src_v.at[b]], rows_v.at[b], gsem[b])

            def group(g, carry2):
                for b in range(_NBUF):
                    j = g * _NBUF + b
                    pltpu.make_async_copy(table.at[src_v.at[j]], rows_v.at[b], gsem[b]).wait()
                    pltpu.sync_copy(rows_v.at[b], acc.at[dst_v.at[j]], add=True)
                    nxt = j + _NBUF

                    @pl.when(nxt < _CHOPS)
                    def _():
                        pltpu.async_copy(table.at[src_v.at[nxt]], rows_v.at[b], gsem[b])

                return carry2

            return lax.fori_loop(0, _NGRP, group, carry)

        lax.fori_loop(0, _SOPS // _CHOPS, outer, 0)

    def combine_phase(qoff):
        def chunk(k, carry):
            row = r0 + k * _RCH
            pltpu.sync_copy(acc.at[pl.ds(row, _RCH), :], cb_a)
            pltpu.sync_copy(q.at[pl.ds(row, _RCH), pl.ds(qoff, 16)], cb_q)
            pltpu.sync_copy(dis2.at[pl.ds(row, _RCH)], cb_d)

            def rowloop(r, carry2):
                d = plsc.load_gather(cb_d, [jnp.full((16,), r, jnp.int32)])
                cb_u[r, :] = d * cb_a[r, :] + cb_q[r, :]
                return carry2

            lax.fori_loop(0, _RCH, rowloop, carry)
            pltpu.sync_copy(cb_u, uS.at[pl.ds(row, _RCH), :])
            return carry

        lax.fori_loop(0, _RPS // _RCH, chunk, 0)
        pltpu.sync_copy(zeros16, acc.at[pl.ds(r0, _RPS), :])

    qc = 16 * c
    pltpu.sync_copy(zeros16, acc.at[pl.ds(r0, _RPS), :])
    pltpu.sync_copy(u1.at[c, pl.ds(r0, _RPS), :], uS.at[pl.ds(r0, _RPS), :])
    plsc.subcore_barrier()
    scatter_phase(uS)
    plsc.subcore_barrier()
    combine_phase(qc)
    plsc.subcore_barrier()
    scatter_phase(uS)
    plsc.subcore_barrier()
    combine_phase(32 + qc)
    plsc.subcore_barrier()
    scatter_phase(uS)
    plsc.subcore_barrier()
    pltpu.sync_copy(acc.at[pl.ds(r0, _RPS), :], out.at[pl.ds(r0, _RPS), pl.ds(qc, 16)])


_mega_call = pl.kernel(
    _mega_body,
    out_type=jax.ShapeDtypeStruct((_NP, 32), jnp.float32),
    mesh=_mesh(),
    compiler_params=pltpu.CompilerParams(use_tc_tiling_on_sc=False,
                                         needs_layout_passes=False),
    scratch_types=[
        pltpu.VMEM((_CHOPS, _LB), jnp.int32),
        pltpu.VMEM((_CHOPS, _LB), jnp.int32),
        pltpu.VMEM((_NBUF, _LB, 16), jnp.float32),
        pltpu.VMEM((_RCH, 16), jnp.float32),
        pltpu.VMEM((_RCH, 16), jnp.float32),
        pltpu.VMEM((_RCH, 16), jnp.float32),
        pltpu.VMEM((_RCH,), jnp.float32),
        pltpu.VMEM_SHARED((_NP, 16), jnp.float32),
        pltpu.VMEM_SHARED((_NP, 16), jnp.float32),
    ] + [pltpu.SemaphoreType.DMA] * _NBUF,
)


def _pb_kernel(x_ref, w_ref, bt_ref, dp_ref,
               r0_ref, q_ref, u1_ref, dis_ref, dis2_ref):
    P = jnp.dot(x_ref[...], w_ref[...], preferred_element_type=jnp.float32)
    deg = dp_ref[0, :, 0:1] + dp_ref[1, :, 0:1]
    dis = jnp.where(deg > 0, lax.rsqrt(jnp.maximum(deg, 1.0)), 0.0)
    r0_ref[...] = P[:, 0:32] + bt_ref[...]
    q_ref[:, 0:32] = dis * P[:, 64:96]
    q_ref[:, 32:64] = dis * P[:, 32:64]
    u1 = dis * P[:, 96:128]
    u1_ref[0] = u1[:, 0:16]
    u1_ref[1] = u1[:, 16:32]
    dis_ref[...] = dis
    dis2_ref[...] = dis * dis


def _pb_call(x, wcat, bt2, degp):
    f32 = jnp.float32
    g = _N // _TN
    return pl.pallas_call(
        _pb_kernel,
        grid=(g,),
        in_specs=[
            pl.BlockSpec((_TN, _IN), lambda i: (i, 0)),
            pl.BlockSpec((_IN, 4 * 32), lambda i: (0, 0)),
            pl.BlockSpec((1, 32), lambda i: (0, 0)),
            pl.BlockSpec((_NC, _TN, _DW), lambda i: (0, i, 0)),
        ],
        out_specs=[
            pl.BlockSpec((_TN, 32), lambda i: (i, 0)),
            pl.BlockSpec((_TN, 64), lambda i: (i, 0)),
            pl.BlockSpec((_NC, _TN, 16), lambda i: (0, i, 0)),
            pl.BlockSpec((_TN, 1), lambda i: (i, 0)),
            pl.BlockSpec((_TN, 1), lambda i: (i, 0)),
        ],
        out_shape=[
            jax.ShapeDtypeStruct((_N, 32), f32),
            jax.ShapeDtypeStruct((_N, 64), f32),
            jax.ShapeDtypeStruct((_NC, _NP, 16), f32),
            jax.ShapeDtypeStruct((_N, 1), f32),
            jax.ShapeDtypeStruct((_N, 1), f32),
        ],
    )(x, wcat, bt2, degp)


def _acc_stats(i, h, s_ref, q_ref):
    s = jnp.sum(h, axis=0, keepdims=True)
    q = jnp.sum(h * h, axis=0, keepdims=True)

    @pl.when(i == 0)
    def _():
        s_ref[...] = s
        q_ref[...] = q

    @pl.when(i > 0)
    def _():
        s_ref[...] += s
        q_ref[...] += q


def _bn_leaky(h, s, q, g, b):
    m = s * (1.0 / _N)
    v = q * (1.0 / _N) - m * m
    hn = (h - m) * lax.rsqrt(v + 1e-5) * g + b
    return jnp.where(hn >= 0, hn, 0.1 * hn)


def _comb3_kernel(pa_ref, dis_ref, r0_ref, h_ref, s_ref, q_ref):
    i = pl.program_id(0)
    h = dis_ref[...] * pa_ref[...] + r0_ref[...]
    h_ref[...] = h
    _acc_stats(i, h, s_ref, q_ref)


def _comb3_call(h3, dis, r0):
    f32 = jnp.float32
    return pl.pallas_call(
        _comb3_kernel,
        grid=(_N // _TN,),
        in_specs=[
            pl.BlockSpec((_TN, 32), lambda i: (i, 0)),
            pl.BlockSpec((_TN, 1), lambda i: (i, 0)),
            pl.BlockSpec((_TN, 32), lambda i: (i, 0)),
        ],
        out_specs=[
            pl.BlockSpec((_TN, 32), lambda i: (i, 0)),
            pl.BlockSpec((1, 32), lambda i: (0, 0)),
            pl.BlockSpec((1, 32), lambda i: (0, 0)),
        ],
        out_shape=[
            jax.ShapeDtypeStruct((_N, 32), f32),
            jax.ShapeDtypeStruct((1, 32), f32),
            jax.ShapeDtypeStruct((1, 32), f32),
        ],
    )(h3, dis, r0)


def _layer_kernel(h_ref, s_ref, q_ref, w_ref, b_ref, g_ref, be_ref,
                  o_ref, so_ref, qo_ref):
    i = pl.program_id(0)
    hn = _bn_leaky(h_ref[...], s_ref[...], q_ref[...], g_ref[...], be_ref[...])
    y = jnp.dot(hn, w_ref[...], preferred_element_type=jnp.float32) + b_ref[...]
    o_ref[...] = y
    _acc_stats(i, y, so_ref, qo_ref)


def _layer_call(h, s, q, w, b, g, be):
    f32 = jnp.float32
    fi, fo = w.shape
    return pl.pallas_call(
        _layer_kernel,
        grid=(_N // _TN,),
        in_specs=[
            pl.BlockSpec((_TN, fi), lambda i: (i, 0)),
            pl.BlockSpec((1, fi), lambda i: (0, 0)),
            pl.BlockSpec((1, fi), lambda i: (0, 0)),
            pl.BlockSpec((fi, fo), lambda i: (0, 0)),
            pl.BlockSpec((1, fo), lambda i: (0, 0)),
            pl.BlockSpec((1, fi), lambda i: (0, 0)),
            pl.BlockSpec((1, fi), lambda i: (0, 0)),
        ],
        out_specs=[
            pl.BlockSpec((_TN, fo), lambda i: (i, 0)),
            pl.BlockSpec((1, fo), lambda i: (0, 0)),
            pl.BlockSpec((1, fo), lambda i: (0, 0)),
        ],
        out_shape=[
            jax.ShapeDtypeStruct((_N, fo), f32),
            jax.ShapeDtypeStruct((1, fo), f32),
            jax.ShapeDtypeStruct((1, fo), f32),
        ],
    )(h, s, q, w, b, g, be)


def _fin_kernel(h_ref, s_ref, q_ref, g_ref, be_ref, out_ref):
    hn = _bn_leaky(h_ref[...], s_ref[...], q_ref[...], g_ref[...], be_ref[...])
    m = jnp.max(hn, axis=1, keepdims=True)
    e = jnp.exp(hn - m)
    out_ref[...] = e / jnp.sum(e, axis=1, keepdims=True)


def _fin_call(h, s, q, g, be):
    return pl.pallas_call(
        _fin_kernel,
        grid=(_N // _TN,),
        in_specs=[
            pl.BlockSpec((_TN, 2), lambda i: (i, 0)),
            pl.BlockSpec((1, 2), lambda i: (0, 0)),
            pl.BlockSpec((1, 2), lambda i: (0, 0)),
            pl.BlockSpec((1, 2), lambda i: (0, 0)),
            pl.BlockSpec((1, 2), lambda i: (0, 0)),
        ],
        out_specs=pl.BlockSpec((_TN, 2), lambda i: (i, 0)),
        out_shape=jax.ShapeDtypeStruct((_N, 2), jnp.float32),
    )(h, s, q, g, be)


def kernel(x, edge_index, W_tag, b_tag, W1, B1, W2, B2, W3, B3, W4, B4,
           g1, be1, g2, be2, g3, be3, g4, be4, g5, be5):
    f32 = jnp.float32
    i32 = jnp.int32
    src = edge_index[0].astype(i32)
    dst = edge_index[1].astype(i32)
    pad = _EPAD - _E
    srcw = jnp.concatenate([src, jnp.zeros((pad,), i32)]).reshape(_TOT, _LB)
    dstw = jnp.concatenate([dst, jnp.full((pad,), _N, i32)]).reshape(_TOT, _LB)
    ones8 = jnp.ones((_LB, _DW), f32)
    zeros8 = jnp.zeros((_RPS, _DW), f32)
    zeros16 = jnp.zeros((_RPS, 16), f32)

    degp = _deg_call(dstw, ones8, zeros8)

    wcat = W_tag.reshape(_K + 1, _IN, 32).transpose(1, 0, 2).reshape(_IN, (_K + 1) * 32)
    bt2 = b_tag.reshape(1, 32)
    r0, qq, u1, dis, dis2 = _pb_call(x, wcat, bt2, degp)

    h3 = _mega_call(u1, srcw, dstw, qq, dis2.reshape(_N), zeros16)

    h, s, q = _comb3_call(h3, dis, r0)
    h, s, q = _layer_call(h, s, q, W1, B1.reshape(1, -1), g1.reshape(1, -1), be1.reshape(1, -1))
    h, s, q = _layer_call(h, s, q, W2, B2.reshape(1, -1), g2.reshape(1, -1), be2.reshape(1, -1))
    h, s, q = _layer_call(h, s, q, W3, B3.reshape(1, -1), g3.reshape(1, -1), be3.reshape(1, -1))
    h, s, q = _layer_call(h, s, q, W4, B4.reshape(1, -1), g4.reshape(1, -1), be4.reshape(1, -1))
    return _fin_call(h, s, q, g5.reshape(1, -1), be5.reshape(1, -1))
